# baseline (speedup 1.0000x reference)
import jax
import jax.numpy as jnp
from jax.experimental import pallas as pl
from jax.experimental.pallas import tpu as pltpu

_B, _V, _E, _NI, _MA = 128, 64, 1024, 1000, 4
_BB1 = 8
_BB2 = 16
_NC = 5
_CHUNK = _NI // _NC
_MLP_STEP = _B // _BB1
_HEAD0 = _MLP_STEP + 1
_ARG0 = _HEAD0 + _NC
_GRID = _ARG0 + _B // _BB2


def _dot_t(x, w):
    return jax.lax.dot_general(x, w, (((1,), (1,)), ((), ())),
                               preferred_element_type=jnp.float32)


def _lrelu(x):
    return jnp.where(x > 0, x, x * 0.01)


def _kernel(value_ref, target_ref, w_arg_blk_ref, b_arg_ref, w_inst_blk_ref,
            b_inst_ref, w1_hbm, b1_ref, w2_hbm, b2_ref, w3_hbm, b3_ref,
            w4_hbm, b4_ref, w5_hbm, b5_ref,
            inst_ref, arg_ref, imm8_ref,
            w1_s, w2_s, w3_s, w4_s, w5_s,
            state_ref, action_ref, qs_ref, vbf_ref, ilog_ref,
            sem1, sem2, sem3, sem4, sem5):
    i = pl.program_id(0)
    cp1 = pltpu.make_async_copy(w1_hbm, w1_s, sem1)
    cp2 = pltpu.make_async_copy(w2_hbm, w2_s, sem2)
    cp3 = pltpu.make_async_copy(w3_hbm, w3_s, sem3)
    cp4 = pltpu.make_async_copy(w4_hbm, w4_s, sem4)
    cp5 = pltpu.make_async_copy(w5_hbm, w5_s, sem5)

    @pl.when(i == 0)
    def _start_weight_dma():
        cp1.start()
        cp2.start()
        cp3.start()
        cp4.start()
        cp5.start()

    @pl.when(i < _MLP_STEP)
    def _sum_phase():
        v = value_ref[...]
        state_ref[pl.ds(i * _BB1, _BB1), :] = jnp.sum(v, axis=1)
        vbf_ref[pl.ds(i * _BB1, _BB1), :, :] = v.astype(jnp.bfloat16)

    @pl.when(i == _MLP_STEP)
    def _mlp_phase():
        cp1.wait()
        cp2.wait()
        cp3.wait()
        state = state_ref[...]
        h = _lrelu(_dot_t(state, w1_s[:, :_E])
                   + _dot_t(target_ref[...], w1_s[:, _E:])
                   + b1_ref[...])
        h = _lrelu(_dot_t(h, w2_s[...]) + b2_ref[...])
        action_ref[...] = _dot_t(h, w3_s[...]) + b3_ref[...]

    @pl.when((i >= _HEAD0) & (i < _ARG0))
    def _head_phase():
        k = i - _HEAD0
        action = action_ref[...]
        ilog_ref[pl.ds(k, 1), :, :] = _dot_t(action, w_inst_blk_ref[...])[None]

        @pl.when(k < _MA)
        def _qs():
            q = _dot_t(action, w_arg_blk_ref[0]) + b_arg_ref[pl.ds(k, 1), :]
            qs_ref[pl.ds(k, 1), :, :] = q[None].astype(jnp.bfloat16)

        @pl.when(k == _NC - 1)
        def _inst_imm8():
            il = jnp.concatenate([ilog_ref[c] for c in range(_NC)],
                                 axis=-1) + b_inst_ref[...]
            il = il - jnp.max(il, axis=-1, keepdims=True)
            ei = jnp.exp(il)
            inst_ref[...] = ei / jnp.sum(ei, axis=-1, keepdims=True)
            cp4.wait()
            cp5.wait()
            h4 = _lrelu(_dot_t(action, w4_s[...]) + b4_ref[...])
            imm8_ref[...] = jax.nn.sigmoid(_dot_t(h4, w5_s[...]) + b5_ref[...])

    @pl.when(i >= _ARG0)
    def _arg_phase():
        j = i - _ARG0
        value_flat = vbf_ref[pl.ds(j * _BB2, _BB2), :, :].reshape(
            _BB2 * _V, _E)
        qs_flat = qs_ref[:, pl.ds(j * _BB2, _BB2), :].reshape(_MA * _BB2, _E)
        ct = jax.lax.dot_general(qs_flat, value_flat, (((1,), (1,)), ((), ())),
                                 preferred_element_type=jnp.float32)
        ct4 = ct.reshape(_MA, _BB2, _BB2, _V)
        bmask = (jax.lax.broadcasted_iota(jnp.int32, (1, _BB2, _BB2, 1), 1) ==
                 jax.lax.broadcasted_iota(jnp.int32, (1, _BB2, _BB2, 1), 2))
        al = jnp.sum(jnp.where(bmask, ct4, 0.0), axis=2)
        al = al - jnp.max(al, axis=-1, keepdims=True)
        ea = jnp.exp(al)
        arg_ref[...] = ea / jnp.sum(ea, axis=-1, keepdims=True)


def _const_spec(*dims):
    n = len(dims)
    return pl.BlockSpec(dims, lambda i, _n=n: (0,) * _n)


def kernel(target, value_embs, W_arg, b_arg, W_inst, b_inst,
           W1, b1, W2, b2, W3, b3, W4, b4, W5, b5):
    inst, argp, imm8 = pl.pallas_call(
        _kernel, grid=(_GRID,),
        in_specs=[
            pl.BlockSpec((_BB1, _V, _E),
                         lambda i: (jnp.minimum(i, _MLP_STEP - 1), 0, 0)),
            _const_spec(_B, _E),
            pl.BlockSpec((1, _E, _E),
                         lambda i: (jnp.clip(i - _HEAD0, 0, _MA - 1), 0, 0)),
            _const_spec(_MA, _E),
            pl.BlockSpec((_CHUNK, _E),
                         lambda i: (jnp.clip(i - _HEAD0, 0, _NC - 1), 0)),
            _const_spec(1, _NI),
            pl.BlockSpec(memory_space=pl.ANY),
            _const_spec(1, _E),
            pl.BlockSpec(memory_space=pl.ANY),
            _const_spec(1, _E),
            pl.BlockSpec(memory_space=pl.ANY),
            _const_spec(1, _E),
            pl.BlockSpec(memory_space=pl.ANY),
            _const_spec(1, _E),
            pl.BlockSpec(memory_space=pl.ANY),
            _const_spec(1, 8),
        ],
        out_specs=[
            _const_spec(_B, _NI),
            pl.BlockSpec((_MA, _BB2, _V),
                         lambda i: (0, jnp.maximum(i - _ARG0, 0), 0)),
            _const_spec(_B, 8),
        ],
        out_shape=[
            jax.ShapeDtypeStruct((_B, _NI), jnp.float32),
            jax.ShapeDtypeStruct((_MA, _B, _V), jnp.float32),
            jax.ShapeDtypeStruct((_B, 8), jnp.float32),
        ],
        scratch_shapes=[
            pltpu.VMEM((_E, 2 * _E), jnp.float32),
            pltpu.VMEM((_E, _E), jnp.float32),
            pltpu.VMEM((_E, _E), jnp.float32),
            pltpu.VMEM((_E, _E), jnp.float32),
            pltpu.VMEM((8, _E), jnp.float32),
            pltpu.VMEM((_B, _E), jnp.float32),
            pltpu.VMEM((_B, _E), jnp.float32),
            pltpu.VMEM((_MA, _B, _E), jnp.bfloat16),
            pltpu.VMEM((_B, _V, _E), jnp.bfloat16),
            pltpu.VMEM((_NC, _B, _CHUNK), jnp.float32),
            pltpu.SemaphoreType.DMA,
            pltpu.SemaphoreType.DMA,
            pltpu.SemaphoreType.DMA,
            pltpu.SemaphoreType.DMA,
            pltpu.SemaphoreType.DMA,
        ],
    )(value_embs, target, W_arg, b_arg, W_inst, b_inst.reshape(1, _NI),
      W1, b1.reshape(1, _E), W2, b2.reshape(1, _E), W3, b3.reshape(1, _E),
      W4, b4.reshape(1, _E), W5, b5.reshape(1, 8))
    return (inst, argp, imm8)


# speedup vs baseline: 1.0500x; 1.0500x over previous
import jax
import jax.numpy as jnp
from jax.experimental import pallas as pl
from jax.experimental.pallas import tpu as pltpu

_B, _V, _E, _NI, _MA = 128, 64, 1024, 1000, 4
_BB1 = 8
_BB2 = 16
_NC = 5
_CHUNK = _NI // _NC
_MLP_STEP = _B // _BB1
_ARG0 = _MLP_STEP + _NC
_GRID = _ARG0 + _B // _BB2


def _dot_t(x, w):
    return jax.lax.dot_general(x, w, (((1,), (1,)), ((), ())),
                               preferred_element_type=jnp.float32)


def _lrelu(x):
    return jnp.where(x > 0, x, x * 0.01)


def _kernel(value_ref, target_ref, w_arg_blk_ref, b_arg_ref, w_inst_blk_ref,
            b_inst_ref, w1_hbm, b1_ref, w2_hbm, b2_ref, w3_hbm, b3_ref,
            w4_hbm, b4_ref, w5_hbm, b5_ref,
            inst_ref, arg_ref, imm8_ref,
            w1_s, w2_s, w3_s, w4_s, w5_s,
            state_ref, action_ref, qs_ref, vbf_ref, ilog_ref,
            sem1, sem2, sem3, sem4, sem5):
    i = pl.program_id(0)
    cp1 = pltpu.make_async_copy(w1_hbm, w1_s, sem1)
    cp2 = pltpu.make_async_copy(w2_hbm, w2_s, sem2)
    cp3 = pltpu.make_async_copy(w3_hbm, w3_s, sem3)
    cp4 = pltpu.make_async_copy(w4_hbm, w4_s, sem4)
    cp5 = pltpu.make_async_copy(w5_hbm, w5_s, sem5)

    @pl.when(i == 0)
    def _start_weight_dma():
        cp1.start()
        cp2.start()
        cp3.start()

    @pl.when(i == _MLP_STEP - 2)
    def _start_tail_weight_dma():
        cp4.start()
        cp5.start()

    @pl.when(i < _MLP_STEP)
    def _sum_phase():
        v = value_ref[...]
        state_ref[pl.ds(i * _BB1, _BB1), :] = jnp.sum(v, axis=1)
        vbf_ref[pl.ds(i * _BB1, _BB1), :, :] = v.astype(jnp.bfloat16)

    @pl.when(i == _MLP_STEP)
    def _mlp_phase():
        cp1.wait()
        cp2.wait()
        cp3.wait()
        state = state_ref[...]
        h = _lrelu(_dot_t(state, w1_s[:, :_E])
                   + _dot_t(target_ref[...], w1_s[:, _E:])
                   + b1_ref[...])
        h = _lrelu(_dot_t(h, w2_s[...]) + b2_ref[...])
        action_ref[...] = _dot_t(h, w3_s[...]) + b3_ref[...]

    @pl.when((i >= _MLP_STEP) & (i < _ARG0))
    def _head_phase():
        k = i - _MLP_STEP
        action = action_ref[...]
        ilog_ref[pl.ds(k, 1), :, :] = _dot_t(action, w_inst_blk_ref[...])[None]

        @pl.when(k < _MA)
        def _qs():
            q = _dot_t(action, w_arg_blk_ref[0]) + b_arg_ref[pl.ds(k, 1), :]
            qs_ref[pl.ds(k, 1), :, :] = q[None].astype(jnp.bfloat16)

        @pl.when(k == _NC - 1)
        def _inst_imm8():
            il = jnp.concatenate([ilog_ref[c] for c in range(_NC)],
                                 axis=-1) + b_inst_ref[...]
            il = il - jnp.max(il, axis=-1, keepdims=True)
            ei = jnp.exp(il)
            inst_ref[...] = ei / jnp.sum(ei, axis=-1, keepdims=True)
            cp4.wait()
            cp5.wait()
            h4 = _lrelu(_dot_t(action, w4_s[...]) + b4_ref[...])
            imm8_ref[...] = jax.nn.sigmoid(_dot_t(h4, w5_s[...]) + b5_ref[...])

    @pl.when(i >= _ARG0)
    def _arg_phase():
        j = i - _ARG0
        value_flat = vbf_ref[pl.ds(j * _BB2, _BB2), :, :].reshape(
            _BB2 * _V, _E)
        qs_flat = qs_ref[:, pl.ds(j * _BB2, _BB2), :].reshape(_MA * _BB2, _E)
        ct = jax.lax.dot_general(qs_flat, value_flat, (((1,), (1,)), ((), ())),
                                 preferred_element_type=jnp.float32)
        ct4 = ct.reshape(_MA, _BB2, _BB2, _V)
        bmask = (jax.lax.broadcasted_iota(jnp.int32, (1, _BB2, _BB2, 1), 1) ==
                 jax.lax.broadcasted_iota(jnp.int32, (1, _BB2, _BB2, 1), 2))
        al = jnp.sum(jnp.where(bmask, ct4, 0.0), axis=2)
        al = al - jnp.max(al, axis=-1, keepdims=True)
        ea = jnp.exp(al)
        arg_ref[...] = ea / jnp.sum(ea, axis=-1, keepdims=True)


def _const_spec(*dims):
    n = len(dims)
    return pl.BlockSpec(dims, lambda i, _n=n: (0,) * _n)


def kernel(target, value_embs, W_arg, b_arg, W_inst, b_inst,
           W1, b1, W2, b2, W3, b3, W4, b4, W5, b5):
    inst, argp, imm8 = pl.pallas_call(
        _kernel, grid=(_GRID,),
        in_specs=[
            pl.BlockSpec((_BB1, _V, _E),
                         lambda i: (jnp.minimum(i, _MLP_STEP - 1), 0, 0)),
            _const_spec(_B, _E),
            pl.BlockSpec((1, _E, _E),
                         lambda i: (jnp.clip(i - _MLP_STEP, 0, _MA - 1), 0, 0)),
            _const_spec(_MA, _E),
            pl.BlockSpec((_CHUNK, _E),
                         lambda i: (jnp.clip(i - _MLP_STEP, 0, _NC - 1), 0)),
            _const_spec(1, _NI),
            pl.BlockSpec(memory_space=pl.ANY),
            _const_spec(1, _E),
            pl.BlockSpec(memory_space=pl.ANY),
            _const_spec(1, _E),
            pl.BlockSpec(memory_space=pl.ANY),
            _const_spec(1, _E),
            pl.BlockSpec(memory_space=pl.ANY),
            _const_spec(1, _E),
            pl.BlockSpec(memory_space=pl.ANY),
            _const_spec(1, 8),
        ],
        out_specs=[
            _const_spec(_B, _NI),
            pl.BlockSpec((_MA, _BB2, _V),
                         lambda i: (0, jnp.maximum(i - _ARG0, 0), 0)),
            _const_spec(_B, 8),
        ],
        out_shape=[
            jax.ShapeDtypeStruct((_B, _NI), jnp.float32),
            jax.ShapeDtypeStruct((_MA, _B, _V), jnp.float32),
            jax.ShapeDtypeStruct((_B, 8), jnp.float32),
        ],
        scratch_shapes=[
            pltpu.VMEM((_E, 2 * _E), jnp.float32),
            pltpu.VMEM((_E, _E), jnp.float32),
            pltpu.VMEM((_E, _E), jnp.float32),
            pltpu.VMEM((_E, _E), jnp.float32),
            pltpu.VMEM((8, _E), jnp.float32),
            pltpu.VMEM((_B, _E), jnp.float32),
            pltpu.VMEM((_B, _E), jnp.float32),
            pltpu.VMEM((_MA, _B, _E), jnp.bfloat16),
            pltpu.VMEM((_B, _V, _E), jnp.bfloat16),
            pltpu.VMEM((_NC, _B, _CHUNK), jnp.float32),
            pltpu.SemaphoreType.DMA,
            pltpu.SemaphoreType.DMA,
            pltpu.SemaphoreType.DMA,
            pltpu.SemaphoreType.DMA,
            pltpu.SemaphoreType.DMA,
        ],
    )(value_embs, target, W_arg, b_arg, W_inst, b_inst.reshape(1, _NI),
      W1, b1.reshape(1, _E), W2, b2.reshape(1, _E), W3, b3.reshape(1, _E),
      W4, b4.reshape(1, _E), W5, b5.reshape(1, 8))
    return (inst, argp, imm8)


# speedup vs baseline: 1.1155x; 1.0624x over previous
import jax
import jax.numpy as jnp
from jax.experimental import pallas as pl
from jax.experimental.pallas import tpu as pltpu

_B, _V, _E, _NI, _MA = 128, 64, 1024, 1000, 4
_BB1 = 16
_BB2 = 16
_NC = 5
_CHUNK = _NI // _NC
_MLP_STEP = _B // _BB1
_ARG0 = _MLP_STEP + _NC
_GRID = _ARG0 + _B // _BB2


def _dot_t(x, w):
    return jax.lax.dot_general(x, w, (((1,), (1,)), ((), ())),
                               preferred_element_type=jnp.float32)


def _lrelu(x):
    return jnp.where(x > 0, x, x * 0.01)


def _kernel(value_ref, target_ref, w_arg_blk_ref, b_arg_ref, w_inst_blk_ref,
            b_inst_ref, w1_hbm, b1_ref, w2_hbm, b2_ref, w3_hbm, b3_ref,
            w4_hbm, b4_ref, w5_hbm, b5_ref,
            inst_ref, arg_ref, imm8_ref,
            w1_s, w2_s, w3_s, w4_s, w5_s,
            state_ref, action_ref, qs_ref, vbf_ref, ilog_ref,
            sem1, sem2, sem3, sem4, sem5):
    i = pl.program_id(0)
    cp1 = pltpu.make_async_copy(w1_hbm, w1_s, sem1)
    cp2 = pltpu.make_async_copy(w2_hbm, w2_s, sem2)
    cp3 = pltpu.make_async_copy(w3_hbm, w3_s, sem3)
    cp4 = pltpu.make_async_copy(w4_hbm, w4_s, sem4)
    cp5 = pltpu.make_async_copy(w5_hbm, w5_s, sem5)

    @pl.when(i == 0)
    def _start_weight_dma():
        cp1.start()
        cp2.start()
        cp3.start()

    @pl.when(i == _MLP_STEP - 2)
    def _start_tail_weight_dma():
        cp4.start()
        cp5.start()

    @pl.when(i < _MLP_STEP)
    def _sum_phase():
        v = value_ref[...]
        state_ref[pl.ds(i * _BB1, _BB1), :] = jnp.sum(v, axis=1)
        vbf_ref[pl.ds(i * _BB1, _BB1), :, :] = v.astype(jnp.bfloat16)

    @pl.when(i == _MLP_STEP)
    def _mlp_phase():
        cp1.wait()
        cp2.wait()
        cp3.wait()
        state = state_ref[...]
        h = _lrelu(_dot_t(state, w1_s[:, :_E])
                   + _dot_t(target_ref[...], w1_s[:, _E:])
                   + b1_ref[...])
        h = _lrelu(_dot_t(h, w2_s[...]) + b2_ref[...])
        action_ref[...] = _dot_t(h, w3_s[...]) + b3_ref[...]

    @pl.when((i >= _MLP_STEP) & (i < _ARG0))
    def _head_phase():
        k = i - _MLP_STEP
        action = action_ref[...]
        ilog_ref[pl.ds(k, 1), :, :] = _dot_t(action, w_inst_blk_ref[...])[None]

        @pl.when(k < _MA)
        def _qs():
            q = _dot_t(action, w_arg_blk_ref[0]) + b_arg_ref[pl.ds(k, 1), :]
            qs_ref[pl.ds(k, 1), :, :] = q[None].astype(jnp.bfloat16)

        @pl.when(k == _NC - 1)
        def _inst_imm8():
            il = jnp.concatenate([ilog_ref[c] for c in range(_NC)],
                                 axis=-1) + b_inst_ref[...]
            il = il - jnp.max(il, axis=-1, keepdims=True)
            ei = jnp.exp(il)
            inst_ref[...] = ei / jnp.sum(ei, axis=-1, keepdims=True)
            cp4.wait()
            cp5.wait()
            h4 = _lrelu(_dot_t(action, w4_s[...]) + b4_ref[...])
            imm8_ref[...] = jax.nn.sigmoid(_dot_t(h4, w5_s[...]) + b5_ref[...])

    @pl.when(i >= _ARG0)
    def _arg_phase():
        j = i - _ARG0
        value_flat = vbf_ref[pl.ds(j * _BB2, _BB2), :, :].reshape(
            _BB2 * _V, _E)
        qs_flat = qs_ref[:, pl.ds(j * _BB2, _BB2), :].reshape(_MA * _BB2, _E)
        ct = jax.lax.dot_general(qs_flat, value_flat, (((1,), (1,)), ((), ())),
                                 preferred_element_type=jnp.float32)
        ct4 = ct.reshape(_MA, _BB2, _BB2, _V)
        bmask = (jax.lax.broadcasted_iota(jnp.int32, (1, _BB2, _BB2, 1), 1) ==
                 jax.lax.broadcasted_iota(jnp.int32, (1, _BB2, _BB2, 1), 2))
        al = jnp.sum(jnp.where(bmask, ct4, 0.0), axis=2)
        al = al - jnp.max(al, axis=-1, keepdims=True)
        ea = jnp.exp(al)
        arg_ref[...] = ea / jnp.sum(ea, axis=-1, keepdims=True)


def _const_spec(*dims):
    n = len(dims)
    return pl.BlockSpec(dims, lambda i, _n=n: (0,) * _n)


def kernel(target, value_embs, W_arg, b_arg, W_inst, b_inst,
           W1, b1, W2, b2, W3, b3, W4, b4, W5, b5):
    inst, argp, imm8 = pl.pallas_call(
        _kernel, grid=(_GRID,),
        in_specs=[
            pl.BlockSpec((_BB1, _V, _E),
                         lambda i: (jnp.minimum(i, _MLP_STEP - 1), 0, 0)),
            _const_spec(_B, _E),
            pl.BlockSpec((1, _E, _E),
                         lambda i: (jnp.clip(i - _MLP_STEP, 0, _MA - 1), 0, 0)),
            _const_spec(_MA, _E),
            pl.BlockSpec((_CHUNK, _E),
                         lambda i: (jnp.clip(i - _MLP_STEP, 0, _NC - 1), 0)),
            _const_spec(1, _NI),
            pl.BlockSpec(memory_space=pl.ANY),
            _const_spec(1, _E),
            pl.BlockSpec(memory_space=pl.ANY),
            _const_spec(1, _E),
            pl.BlockSpec(memory_space=pl.ANY),
            _const_spec(1, _E),
            pl.BlockSpec(memory_space=pl.ANY),
            _const_spec(1, _E),
            pl.BlockSpec(memory_space=pl.ANY),
            _const_spec(1, 8),
        ],
        out_specs=[
            _const_spec(_B, _NI),
            pl.BlockSpec((_MA, _BB2, _V),
                         lambda i: (0, jnp.maximum(i - _ARG0, 0), 0)),
            _const_spec(_B, 8),
        ],
        out_shape=[
            jax.ShapeDtypeStruct((_B, _NI), jnp.float32),
            jax.ShapeDtypeStruct((_MA, _B, _V), jnp.float32),
            jax.ShapeDtypeStruct((_B, 8), jnp.float32),
        ],
        scratch_shapes=[
            pltpu.VMEM((_E, 2 * _E), jnp.float32),
            pltpu.VMEM((_E, _E), jnp.float32),
            pltpu.VMEM((_E, _E), jnp.float32),
            pltpu.VMEM((_E, _E), jnp.float32),
            pltpu.VMEM((8, _E), jnp.float32),
            pltpu.VMEM((_B, _E), jnp.float32),
            pltpu.VMEM((_B, _E), jnp.float32),
            pltpu.VMEM((_MA, _B, _E), jnp.bfloat16),
            pltpu.VMEM((_B, _V, _E), jnp.bfloat16),
            pltpu.VMEM((_NC, _B, _CHUNK), jnp.float32),
            pltpu.SemaphoreType.DMA,
            pltpu.SemaphoreType.DMA,
            pltpu.SemaphoreType.DMA,
            pltpu.SemaphoreType.DMA,
            pltpu.SemaphoreType.DMA,
        ],
    )(value_embs, target, W_arg, b_arg, W_inst, b_inst.reshape(1, _NI),
      W1, b1.reshape(1, _E), W2, b2.reshape(1, _E), W3, b3.reshape(1, _E),
      W4, b4.reshape(1, _E), W5, b5.reshape(1, 8))
    return (inst, argp, imm8)
